# baseline (speedup 1.0000x reference)
import functools

import jax
import jax.numpy as jnp
from jax import lax
from jax.experimental import pallas as pl
from jax.experimental.pallas import tpu as pltpu
from jax.experimental.pallas import tpu_sc as plsc

VOCAB = 1000
VPAD = 1024
N_TOK = 1024 * 50
NW = 32
ROWS_PER_W = N_TOK // NW
G = 16
GROUPS = ROWS_PER_W // G
NBUF = 4


def _lse_body(table_ref, out_ref):
    x = table_ref[...]
    m = jnp.max(x, axis=1, keepdims=True)
    s = jnp.sum(jnp.exp(x - m), axis=1, keepdims=True)
    out_ref[...] = m + jnp.log(s)


def _sc_body(table_hbm, idx_hbm, tgt_hbm, lse_hbm,
             out_hbm, part_hbm,
             idx_v, tgt_v, lse_v, r0, r1, r2, r3, acc_v,
             sg0, sg1, sg2, sg3, so0, so1, so2, so3):
    rows = (r0, r1, r2, r3)
    semg = (sg0, sg1, sg2, sg3)
    semo = (so0, so1, so2, so3)
    wid = lax.axis_index("s") * 2 + lax.axis_index("c")
    base = wid * ROWS_PER_W

    pltpu.sync_copy(idx_hbm.at[pl.ds(base, ROWS_PER_W)], idx_v)
    pltpu.sync_copy(tgt_hbm.at[pl.ds(base, ROWS_PER_W)], tgt_v)
    pltpu.sync_copy(lse_hbm, lse_v)

    iota = lax.iota(jnp.int32, G)

    def start_gather(g, b):
        idx16 = idx_v[pl.ds(g * G, G)]
        pltpu.async_copy(table_hbm.at[idx16], rows[b], semg[b])

    def wait_gather(b):
        pltpu.make_async_copy(table_hbm.at[iota], rows[b], semg[b]).wait()

    def start_out(g, b):
        pltpu.async_copy(rows[b], out_hbm.at[pl.ds(base + g * G, G)], semo[b])

    def wait_out(b):
        pltpu.make_async_copy(rows[b], out_hbm.at[pl.ds(0, G)],
                              semo[b]).wait()

    for b in range(NBUF):
        start_gather(b, b)

    def body(k, acc):
        for b in range(NBUF):
            g = k * NBUF + b
            wait_gather(b)
            idx16 = idx_v[pl.ds(g * G, G)]
            tg16 = tgt_v[pl.ds(g * G, G)]
            vals = plsc.load_gather(rows[b], [iota, tg16])
            lsev = plsc.load_gather(lse_v, [idx16])
            acc = acc + (vals - lsev)
            start_out(g, b)

            @pl.when(k + 1 < GROUPS // NBUF)
            def _():
                wait_out(b)
                start_gather(g + NBUF, b)

        return acc

    acc = lax.fori_loop(0, GROUPS // NBUF, body, jnp.zeros((G,), jnp.float32))
    for b in range(NBUF):
        wait_out(b)
    acc_v[...] = acc
    pltpu.sync_copy(acc_v, part_hbm.at[pl.ds(wid * G, G)])


def _final_body(part_ref, out_ref):
    out_ref[...] = jnp.full((1, 1), -jnp.sum(part_ref[...]) / N_TOK,
                            jnp.float32)


SLICE_BLK = 1024


def _slice_body(in_ref, out_ref):
    out_ref[...] = in_ref[:, :VOCAB]


@jax.jit
def kernel(idx, expected, table):
    idx_f = idx.reshape(-1)
    tgt_f = expected.reshape(-1)

    lse = pl.pallas_call(
        _lse_body,
        out_shape=jax.ShapeDtypeStruct((VOCAB, 1), jnp.float32),
    )(table)

    sc = pl.kernel(
        _sc_body,
        out_type=(
            jax.ShapeDtypeStruct((N_TOK, VOCAB), jnp.float32),
            jax.ShapeDtypeStruct((NW * G,), jnp.float32),
        ),
        mesh=plsc.VectorSubcoreMesh(core_axis_name="c", subcore_axis_name="s"),
        compiler_params=pltpu.CompilerParams(
            needs_layout_passes=False, use_tc_tiling_on_sc=False),
        scratch_types=(
            pltpu.VMEM((ROWS_PER_W,), jnp.int32),
            pltpu.VMEM((ROWS_PER_W,), jnp.int32),
            pltpu.VMEM((VOCAB,), jnp.float32),
            pltpu.VMEM((G, VOCAB), jnp.float32),
            pltpu.VMEM((G, VOCAB), jnp.float32),
            pltpu.VMEM((G, VOCAB), jnp.float32),
            pltpu.VMEM((G, VOCAB), jnp.float32),
            pltpu.VMEM((G,), jnp.float32),
            pltpu.SemaphoreType.DMA,
            pltpu.SemaphoreType.DMA,
            pltpu.SemaphoreType.DMA,
            pltpu.SemaphoreType.DMA,
            pltpu.SemaphoreType.DMA,
            pltpu.SemaphoreType.DMA,
            pltpu.SemaphoreType.DMA,
            pltpu.SemaphoreType.DMA,
        ),
    )
    logits2, partials = sc(table, idx_f, tgt_f, lse.reshape(-1))

    cost = pl.pallas_call(
        _final_body,
        out_shape=jax.ShapeDtypeStruct((1, 1), jnp.float32),
    )(partials)

    return (logits2, cost[0, 0])


# speedup vs baseline: 1.6559x; 1.6559x over previous
import functools

import jax
import jax.numpy as jnp
from jax import lax
from jax.experimental import pallas as pl
from jax.experimental.pallas import tpu as pltpu
from jax.experimental.pallas import tpu_sc as plsc

VOCAB = 1000
VPAD = 1024
N_TOK = 1024 * 50
NW = 32
ROWS_PER_W = N_TOK // NW
G = 16
GROUPS = ROWS_PER_W // G
NBUF = 4


def _lse_body(table_ref, out_ref):
    x = table_ref[...]
    m = jnp.max(x, axis=1, keepdims=True)
    s = jnp.sum(jnp.exp(x - m), axis=1, keepdims=True)
    out_ref[...] = m + jnp.log(s)


def _sc_body(table_hbm, idx_hbm, tgt_hbm, lse_hbm,
             out_hbm, part_hbm,
             idx_v, tgt_v, lse_v, r0, r1, r2, r3, acc_v,
             sg0, sg1, sg2, sg3, so0, so1, so2, so3):
    rows = (r0, r1, r2, r3)
    semg = (sg0, sg1, sg2, sg3)
    semo = (so0, so1, so2, so3)
    wid = lax.axis_index("s") * 2 + lax.axis_index("c")
    base = wid * ROWS_PER_W

    pltpu.sync_copy(idx_hbm.at[pl.ds(base, ROWS_PER_W)], idx_v)
    pltpu.sync_copy(tgt_hbm.at[pl.ds(base, ROWS_PER_W)], tgt_v)
    pltpu.sync_copy(lse_hbm, lse_v)

    iota = lax.iota(jnp.int32, G)

    def start_gather(g, b):
        idx16 = idx_v[pl.ds(g * G, G)]
        pltpu.async_copy(table_hbm.at[idx16], rows[b], semg[b])

    def wait_gather(b):
        pltpu.make_async_copy(table_hbm.at[iota], rows[b], semg[b]).wait()

    def start_out(g, b):
        pltpu.async_copy(rows[b], out_hbm.at[pl.ds(base + g * G, G)], semo[b])

    def wait_out(b):
        pltpu.make_async_copy(rows[b], out_hbm.at[pl.ds(0, G)],
                              semo[b]).wait()

    for b in range(NBUF):
        start_gather(b, b)

    def body(k, acc):
        for b in range(NBUF):
            g = k * NBUF + b
            wait_gather(b)
            idx16 = idx_v[pl.ds(g * G, G)]
            tg16 = tgt_v[pl.ds(g * G, G)]
            vals = plsc.load_gather(rows[b], [iota, tg16])
            lsev = plsc.load_gather(lse_v, [idx16])
            acc = acc + (vals - lsev)
            start_out(g, b)

            @pl.when(k + 1 < GROUPS // NBUF)
            def _():
                wait_out(b)
                start_gather(g + NBUF, b)

        return acc

    acc = lax.fori_loop(0, GROUPS // NBUF, body, jnp.zeros((G,), jnp.float32))
    for b in range(NBUF):
        wait_out(b)
    acc_v[...] = acc
    pltpu.sync_copy(acc_v, part_hbm.at[pl.ds(wid * G, G)])


def _final_body(part_ref, out_ref):
    out_ref[...] = jnp.full((1, 1), -jnp.sum(part_ref[...]) / N_TOK,
                            jnp.float32)


SLICE_BLK = 1024


def _slice_body(in_ref, out_ref):
    out_ref[...] = in_ref[:, :VOCAB]


@jax.jit
def kernel(idx, expected, table):
    idx_f = idx.reshape(-1)
    tgt_f = expected.reshape(-1)
    table_pad = jnp.pad(table, ((0, 0), (0, VPAD - VOCAB)))

    lse = pl.pallas_call(
        _lse_body,
        out_shape=jax.ShapeDtypeStruct((VOCAB, 1), jnp.float32),
    )(table)

    sc = pl.kernel(
        _sc_body,
        out_type=(
            jax.ShapeDtypeStruct((N_TOK, VPAD), jnp.float32),
            jax.ShapeDtypeStruct((NW * G,), jnp.float32),
        ),
        mesh=plsc.VectorSubcoreMesh(core_axis_name="c", subcore_axis_name="s"),
        compiler_params=pltpu.CompilerParams(needs_layout_passes=False),
        scratch_types=(
            pltpu.VMEM((ROWS_PER_W,), jnp.int32),
            pltpu.VMEM((ROWS_PER_W,), jnp.int32),
            pltpu.VMEM((VOCAB,), jnp.float32),
            pltpu.VMEM((G, VPAD), jnp.float32),
            pltpu.VMEM((G, VPAD), jnp.float32),
            pltpu.VMEM((G, VPAD), jnp.float32),
            pltpu.VMEM((G, VPAD), jnp.float32),
            pltpu.VMEM((G,), jnp.float32),
            pltpu.SemaphoreType.DMA,
            pltpu.SemaphoreType.DMA,
            pltpu.SemaphoreType.DMA,
            pltpu.SemaphoreType.DMA,
            pltpu.SemaphoreType.DMA,
            pltpu.SemaphoreType.DMA,
            pltpu.SemaphoreType.DMA,
            pltpu.SemaphoreType.DMA,
        ),
    )
    logits_pad, partials = sc(table_pad, idx_f, tgt_f, lse.reshape(-1))
    logits2 = logits_pad[:, :VOCAB]

    cost = pl.pallas_call(
        _final_body,
        out_shape=jax.ShapeDtypeStruct((1, 1), jnp.float32),
    )(partials)

    return (logits2, cost[0, 0])
